# baseline (speedup 1.0000x reference)
import functools

import jax
import jax.numpy as jnp
from jax import lax
from jax.experimental import pallas as pl
from jax.experimental.pallas import tpu as pltpu
from jax.experimental.pallas import tpu_sc as plsc

D = 128
K = 128
NC = 2
NS = 16
NW = NC * NS
V = 16

_MESH = plsc.VectorSubcoreMesh(
    core_axis_name="c", subcore_axis_name="s", num_cores=NC, num_subcores=NS
)


def _make_sc_count(EC, Npad):
  RPT = Npad // NS

  @functools.partial(
      pl.kernel,
      out_type=jax.ShapeDtypeStruct((NC * Npad, 128), jnp.float32),
      mesh=_MESH,
      scratch_types=[
          pltpu.VMEM((EC, K), jnp.int32),
          pltpu.VMEM((K, 128), jnp.float32),
          pltpu.VMEM_SHARED((Npad, 128), jnp.float32),
      ],
  )
  def count_kernel(dstc, ones_hbm, zeros_hbm, cnt_out, didx_all, ones_v, acc):
    c = lax.axis_index("c")
    s = lax.axis_index("s")
    g = c * NS + s
    pltpu.sync_copy(dstc.at[g], didx_all)
    pltpu.sync_copy(ones_hbm, ones_v)
    rows = pl.ds(s * RPT, RPT)
    pltpu.sync_copy(zeros_hbm.at[rows], acc.at[rows])
    plsc.subcore_barrier()

    def step(t, carry):
      pltpu.sync_copy(ones_v, acc.at[didx_all.at[t]], add=True)
      return carry

    lax.fori_loop(0, EC, step, 0)
    plsc.subcore_barrier()
    pltpu.sync_copy(acc.at[rows], cnt_out.at[pl.ds(c * Npad + s * RPT, RPT)])

  return count_kernel


def _make_sc_scatter(EC, Npad):
  RPT = Npad // NS

  EC2 = EC // 2

  @functools.partial(
      pl.kernel,
      out_type=jax.ShapeDtypeStruct((NC * Npad, D), jnp.float32),
      mesh=_MESH,
      scratch_types=[
          pltpu.VMEM((EC2, K), jnp.int32),
          pltpu.VMEM((EC2, K), jnp.int32),
          pltpu.VMEM((K, D), jnp.float32),
          pltpu.VMEM((K, D), jnp.float32),
          pltpu.VMEM_SHARED((Npad, D), jnp.float32),
          pltpu.SemaphoreType.DMA,
          pltpu.SemaphoreType.DMA,
      ],
  )
  def scatter_kernel(h_hbm, srcc, dstc, zeros_hbm, part_out,
                     sidx_all, didx_all, r0, r1, acc, sem0, sem1):
    c = lax.axis_index("c")
    s = lax.axis_index("s")
    g = c * NS + s
    rows = pl.ds(s * RPT, RPT)
    pltpu.sync_copy(zeros_hbm.at[rows], acc.at[rows])
    plsc.subcore_barrier()

    for p in range(2):
      pltpu.sync_copy(srcc.at[g, pl.ds(p * EC2, EC2)], sidx_all)
      pltpu.sync_copy(dstc.at[g, pl.ds(p * EC2, EC2)], didx_all)
      pltpu.async_copy(h_hbm.at[sidx_all.at[0]], r0, sem0)

      def step(u, carry):
        t0 = 2 * u
        pltpu.async_copy(h_hbm.at[sidx_all.at[t0 + 1]], r1, sem1)
        pltpu.make_async_copy(h_hbm.at[sidx_all.at[t0]], r0, sem0).wait()
        pltpu.sync_copy(r0, acc.at[didx_all.at[t0]], add=True)

        @pl.when(t0 + 2 < EC2)
        def _next():
          pltpu.async_copy(h_hbm.at[sidx_all.at[t0 + 2]], r0, sem0)

        pltpu.make_async_copy(h_hbm.at[sidx_all.at[t0 + 1]], r1, sem1).wait()
        pltpu.sync_copy(r1, acc.at[didx_all.at[t0 + 1]], add=True)
        return carry

      lax.fori_loop(0, EC2 // 2, step, 0)

    plsc.subcore_barrier()
    pltpu.sync_copy(acc.at[rows], part_out.at[pl.ds(c * Npad + s * RPT, RPT)])

  return scatter_kernel


def _row_spec(B):
  return pl.BlockSpec((B, D), lambda i: (i, 0))


def _col_spec(B):
  return pl.BlockSpec((B, 1), lambda i: (i, 0))


def _w_spec():
  return pl.BlockSpec((D, D), lambda i: (0, 0))


def _b_spec():
  return pl.BlockSpec((1, D), lambda i: (0, 0))


def _tc1_body(c0, c1, x, w, hs, dis, icnt):
  cnt = c0[...] + c1[...]
  d = lax.rsqrt(cnt + 1.0)
  hs[...] = jnp.dot(x[...], w[...], preferred_element_type=jnp.float32) * d
  dis[...] = d
  icnt[...] = 1.0 / jnp.maximum(cnt, 1.0)


def _tc2_body(p0, p1, hs, dis, b, z1):
  z1[...] = jnp.maximum((p0[...] + p1[...] + hs[...]) * dis[...] + b[...], 0.0)


def _tc3_body(q0, q1, icnt, z1, dis, wl, bl, wr, wout, hs3):
  mean = (q0[...] + q1[...]) * icnt[...]
  z2 = jnp.maximum(
      jnp.dot(mean, wl[...], preferred_element_type=jnp.float32) + bl[...]
      + jnp.dot(z1[...], wr[...], preferred_element_type=jnp.float32), 0.0)
  hs3[...] = jnp.dot(z2, wout[...], preferred_element_type=jnp.float32) * dis[...]


def _tc4_body(r0, r1, hs3, dis, b, out):
  out[...] = (r0[...] + r1[...] + hs3[...]) * dis[...] + b[...]


def kernel(x, edge_index, W1, b1, Wl, bl, Wr, Wout, bout):
  N = x.shape[0]
  E = edge_index.shape[1]
  f32 = jnp.float32

  Npad = ((N + 1 + 2047) // 2048) * 2048
  EC = (E + K * NW - 1) // (K * NW)
  EC = ((EC + 3) // 4) * 4
  Epad = EC * K * NW
  CH = NW * EC

  src = edge_index[0]
  dst = edge_index[1]
  pad = jnp.full((Epad - E,), N, dtype=jnp.int32)
  srcc = jnp.concatenate([src, pad]).reshape(NW, EC, K)
  dstc = jnp.concatenate([dst, pad]).reshape(NW, EC, K)

  xp = jnp.zeros((Npad, D), f32).at[:N].set(x)
  zerosd = jnp.zeros((Npad, D), f32)
  onesk = jnp.ones((K, 128), f32)

  sc_count = _make_sc_count(EC, Npad)
  sc_scatter = _make_sc_scatter(EC, Npad)

  cnt_parts = sc_count(dstc, onesk, zerosd)
  c0 = cnt_parts[:Npad, :1]
  c1 = cnt_parts[Npad:, :1]

  B = Npad // 16
  grid = (Npad // B,)

  hs1, dis, icnt = pl.pallas_call(
      _tc1_body,
      grid=grid,
      in_specs=[_col_spec(B), _col_spec(B), _row_spec(B), _w_spec()],
      out_specs=[_row_spec(B), _col_spec(B), _col_spec(B)],
      out_shape=[
          jax.ShapeDtypeStruct((Npad, D), f32),
          jax.ShapeDtypeStruct((Npad, 1), f32),
          jax.ShapeDtypeStruct((Npad, 1), f32),
      ],
  )(c0, c1, xp, W1)

  p = sc_scatter(hs1, srcc, dstc, zerosd)
  z1 = pl.pallas_call(
      _tc2_body,
      grid=grid,
      in_specs=[_row_spec(B), _row_spec(B), _row_spec(B), _col_spec(B),
                _b_spec()],
      out_specs=_row_spec(B),
      out_shape=jax.ShapeDtypeStruct((Npad, D), f32),
  )(p[:Npad], p[Npad:], hs1, dis, b1.reshape(1, D))

  q = sc_scatter(z1, srcc, dstc, zerosd)
  hs3 = pl.pallas_call(
      _tc3_body,
      grid=grid,
      in_specs=[_row_spec(B), _row_spec(B), _col_spec(B), _row_spec(B),
                _col_spec(B), _w_spec(), _b_spec(), _w_spec(), _w_spec()],
      out_specs=_row_spec(B),
      out_shape=jax.ShapeDtypeStruct((Npad, D), f32),
  )(q[:Npad], q[Npad:], icnt, z1, dis, Wl, bl.reshape(1, D), Wr, Wout)

  r = sc_scatter(hs3, srcc, dstc, zerosd)
  out = pl.pallas_call(
      _tc4_body,
      grid=grid,
      in_specs=[_row_spec(B), _row_spec(B), _row_spec(B), _col_spec(B),
                _b_spec()],
      out_specs=_row_spec(B),
      out_shape=jax.ShapeDtypeStruct((Npad, D), f32),
  )(r[:Npad], r[Npad:], hs3, dis, bout.reshape(1, D))

  return out[:N]


# speedup vs baseline: 1.2696x; 1.2696x over previous
import functools

import jax
import jax.numpy as jnp
from jax import lax
from jax.experimental import pallas as pl
from jax.experimental.pallas import tpu as pltpu
from jax.experimental.pallas import tpu_sc as plsc

D = 128
K = 128
NC = 2
NS = 16
NW = NC * NS
V = 16

_MESH = plsc.VectorSubcoreMesh(
    core_axis_name="c", subcore_axis_name="s", num_cores=NC, num_subcores=NS
)


def _make_sc_count(EC, Npad):
  RPT = Npad // NS

  @functools.partial(
      pl.kernel,
      out_type=jax.ShapeDtypeStruct((NC * Npad, 128), jnp.float32),
      mesh=_MESH,
      scratch_types=[
          pltpu.VMEM((EC, K), jnp.int32),
          pltpu.VMEM((K, 128), jnp.float32),
          pltpu.VMEM_SHARED((Npad, 128), jnp.float32),
      ],
  )
  def count_kernel(dstc, ones_hbm, zeros_hbm, cnt_out, didx_all, ones_v, acc):
    c = lax.axis_index("c")
    s = lax.axis_index("s")
    g = c * NS + s
    pltpu.sync_copy(dstc.at[pl.ds(g * EC, EC)], didx_all)
    pltpu.sync_copy(ones_hbm, ones_v)
    rows = pl.ds(s * RPT, RPT)
    pltpu.sync_copy(zeros_hbm.at[rows], acc.at[rows])
    plsc.subcore_barrier()

    def step(t, carry):
      pltpu.sync_copy(ones_v, acc.at[didx_all.at[t]], add=True)
      return carry

    lax.fori_loop(0, EC, step, 0)
    plsc.subcore_barrier()
    pltpu.sync_copy(acc.at[rows], cnt_out.at[pl.ds(c * Npad + s * RPT, RPT)])

  return count_kernel


PH = 16


def _make_sc_scatter(EC0, EC1, Npad):
  RPT = Npad // NS
  ncores = 1 if EC1 == 0 else NC

  @functools.partial(
      pl.kernel,
      out_type=jax.ShapeDtypeStruct((ncores * Npad, D), jnp.float32),
      mesh=_MESH,
      scratch_types=[
          pltpu.VMEM((PH, K), jnp.int32),
          pltpu.VMEM((PH, K), jnp.int32),
          pltpu.VMEM((K, D), jnp.float32),
          pltpu.VMEM((K, D), jnp.float32),
          pltpu.VMEM_SHARED((Npad, D), jnp.float32),
          pltpu.SemaphoreType.DMA,
          pltpu.SemaphoreType.DMA,
      ],
  )
  def scatter_kernel(h_hbm, srcc, dstc, zeros_hbm, part_out,
                     sidx, didx, r0, r1, acc, sem0, sem1):
    c = lax.axis_index("c")
    s = lax.axis_index("s")

    def run(tstart, nph):
      for p in range(nph):
        base = tstart + p * PH
        pltpu.sync_copy(srcc.at[pl.ds(base, PH)], sidx)
        pltpu.sync_copy(dstc.at[pl.ds(base, PH)], didx)
        pltpu.async_copy(h_hbm.at[sidx.at[0]], r0, sem0)

        def step(u, carry):
          t0 = 2 * u
          pltpu.async_copy(h_hbm.at[sidx.at[t0 + 1]], r1, sem1)
          pltpu.make_async_copy(h_hbm.at[sidx.at[t0]], r0, sem0).wait()
          pltpu.sync_copy(r0, acc.at[didx.at[t0]], add=True)

          @pl.when(t0 + 2 < PH)
          def _next():
            pltpu.async_copy(h_hbm.at[sidx.at[t0 + 2]], r0, sem0)

          pltpu.make_async_copy(h_hbm.at[sidx.at[t0 + 1]], r1, sem1).wait()
          pltpu.sync_copy(r1, acc.at[didx.at[t0 + 1]], add=True)
          return carry

        lax.fori_loop(0, PH // 2, step, 0)

    if ncores == 1:

      @pl.when(c == 0)
      def _sc0_only():
        rows = pl.ds(s * RPT, RPT)
        pltpu.sync_copy(zeros_hbm.at[rows], acc.at[rows])
        plsc.subcore_barrier()
        run(s * EC0, EC0 // PH)
        plsc.subcore_barrier()
        pltpu.sync_copy(acc.at[rows], part_out.at[rows])

    else:
      rows = pl.ds(s * RPT, RPT)
      pltpu.sync_copy(zeros_hbm.at[rows], acc.at[rows])
      plsc.subcore_barrier()

      @pl.when(c == 0)
      def _sc0():
        run(s * EC0, EC0 // PH)

      @pl.when(c == 1)
      def _sc1():
        run(NS * EC0 + s * EC1, EC1 // PH)

      plsc.subcore_barrier()
      pltpu.sync_copy(acc.at[rows], part_out.at[pl.ds(c * Npad + s * RPT, RPT)])

  return scatter_kernel


def _row_spec(B):
  return pl.BlockSpec((B, D), lambda i: (i, 0))


def _col_spec(B):
  return pl.BlockSpec((B, 1), lambda i: (i, 0))


def _w_spec():
  return pl.BlockSpec((D, D), lambda i: (0, 0))


def _b_spec():
  return pl.BlockSpec((1, D), lambda i: (0, 0))


def _tc1_body(c0, c1, x, w, hs, dis, icnt):
  cnt = c0[...] + c1[...]
  d = lax.rsqrt(cnt + 1.0)
  hs[...] = jnp.dot(x[...], w[...], preferred_element_type=jnp.float32) * d
  dis[...] = d
  icnt[...] = 1.0 / jnp.maximum(cnt, 1.0)


def _psum(parts):
  acc = parts[0][...]
  for p in parts[1:]:
    acc = acc + p[...]
  return acc


def _tc2_factory(npart):
  def body(*refs):
    parts, (hs, dis, b, z1) = refs[:npart], refs[npart:]
    z1[...] = jnp.maximum((_psum(parts) + hs[...]) * dis[...] + b[...], 0.0)
  return body


def _tc3_factory(npart):
  def body(*refs):
    parts, (icnt, z1, dis, wl, bl, wr, wout, hs3) = refs[:npart], refs[npart:]
    mean = _psum(parts) * icnt[...]
    z2 = jnp.maximum(
        jnp.dot(mean, wl[...], preferred_element_type=jnp.float32) + bl[...]
        + jnp.dot(z1[...], wr[...], preferred_element_type=jnp.float32), 0.0)
    hs3[...] = jnp.dot(z2, wout[...],
                       preferred_element_type=jnp.float32) * dis[...]
  return body


def _tc4_factory(npart):
  def body(*refs):
    parts, (hs3, dis, b, out) = refs[:npart], refs[npart:]
    out[...] = (_psum(parts) + hs3[...]) * dis[...] + b[...]
  return body


_SC0_FRAC = 0.875


def kernel(x, edge_index, W1, b1, Wl, bl, Wr, Wout, bout):
  N = x.shape[0]
  E = edge_index.shape[1]
  f32 = jnp.float32

  Npad = ((N + 1 + 2047) // 2048) * 2048
  ECt = (E + K * NS - 1) // (K * NS)
  ECt = ((ECt + PH - 1) // PH) * PH
  Epad = ECt * K * NS
  CH = NS * ECt
  ECc = ECt // 2

  units = ECt // PH
  u0 = min(units, max(1, round(units * _SC0_FRAC)))
  EC0, EC1 = u0 * PH, (units - u0) * PH
  npart = 1 if EC1 == 0 else 2

  src = edge_index[0]
  dst = edge_index[1]
  pad = jnp.full((Epad - E,), N, dtype=jnp.int32)
  srcc = jnp.concatenate([src, pad]).reshape(CH, K)
  dstc = jnp.concatenate([dst, pad]).reshape(CH, K)

  xp = jnp.zeros((Npad, D), f32).at[:N].set(x)
  zerosd = jnp.zeros((Npad, D), f32)
  onesk = jnp.ones((K, 128), f32)

  sc_count = _make_sc_count(ECc, Npad)
  sc_scatter = _make_sc_scatter(EC0, EC1, Npad)

  def parts(a):
    return [a[i * Npad:(i + 1) * Npad] for i in range(npart)]

  cnt_parts = sc_count(dstc, onesk, zerosd)
  c0 = cnt_parts[:Npad, :1]
  c1 = cnt_parts[Npad:, :1]

  B = Npad // 16
  grid = (Npad // B,)
  prow = [_row_spec(B)] * npart

  hs1, dis, icnt = pl.pallas_call(
      _tc1_body,
      grid=grid,
      in_specs=[_col_spec(B), _col_spec(B), _row_spec(B), _w_spec()],
      out_specs=[_row_spec(B), _col_spec(B), _col_spec(B)],
      out_shape=[
          jax.ShapeDtypeStruct((Npad, D), f32),
          jax.ShapeDtypeStruct((Npad, 1), f32),
          jax.ShapeDtypeStruct((Npad, 1), f32),
      ],
  )(c0, c1, xp, W1)

  p = sc_scatter(hs1, srcc, dstc, zerosd)
  z1 = pl.pallas_call(
      _tc2_factory(npart),
      grid=grid,
      in_specs=prow + [_row_spec(B), _col_spec(B), _b_spec()],
      out_specs=_row_spec(B),
      out_shape=jax.ShapeDtypeStruct((Npad, D), f32),
  )(*parts(p), hs1, dis, b1.reshape(1, D))

  q = sc_scatter(z1, srcc, dstc, zerosd)
  hs3 = pl.pallas_call(
      _tc3_factory(npart),
      grid=grid,
      in_specs=prow + [_col_spec(B), _row_spec(B), _col_spec(B), _w_spec(),
                       _b_spec(), _w_spec(), _w_spec()],
      out_specs=_row_spec(B),
      out_shape=jax.ShapeDtypeStruct((Npad, D), f32),
  )(*parts(q), icnt, z1, dis, Wl, bl.reshape(1, D), Wr, Wout)

  r = sc_scatter(hs3, srcc, dstc, zerosd)
  out = pl.pallas_call(
      _tc4_factory(npart),
      grid=grid,
      in_specs=prow + [_row_spec(B), _col_spec(B), _b_spec()],
      out_specs=_row_spec(B),
      out_shape=jax.ShapeDtypeStruct((Npad, D), f32),
  )(*parts(r), hs3, dis, bout.reshape(1, D))

  return out[:N]
